# baseline (speedup 1.0000x reference)
import jax
import jax.numpy as jnp
from jax.experimental import pallas as pl
from jax.experimental.pallas import tpu as pltpu

_N, _L, _C, _H, _G = 50000, 9, 256, 512, 256
_BLK = 512
_NBLK = (_N + _BLK - 1) // _BLK
_NPAD = _NBLK * _BLK


def _mlp_energy_body(b3_ref, x_ref, bidx_ref, w1_ref, b1_ref, w2_ref, b2_ref,
                     w3_ref, out_ref):
    i = pl.program_id(0)
    x = x_ref[:, :].astype(jnp.bfloat16)
    h = jnp.dot(x, w1_ref[:].astype(jnp.bfloat16),
                preferred_element_type=jnp.float32) + b1_ref[:]
    h = h * jax.nn.sigmoid(h)
    h = jnp.dot(h.astype(jnp.bfloat16), w2_ref[:].astype(jnp.bfloat16),
                preferred_element_type=jnp.float32) + b2_ref[:]
    h = h * jax.nn.sigmoid(h)
    e = jnp.sum(h * w3_ref[:], axis=1) + b3_ref[0]
    rows = i * _BLK + jax.lax.broadcasted_iota(jnp.int32, (_BLK,), 0)
    e = jnp.where(rows < _N, e, 0.0)
    idx = bidx_ref[0, 0, :]
    onehot = (idx[:, None] == jax.lax.broadcasted_iota(
        jnp.int32, (_BLK, _G), 1)).astype(jnp.float32)
    part = jnp.dot(e[None, :], onehot, preferred_element_type=jnp.float32)

    @pl.when(i == 0)
    def _():
        out_ref[:] = jnp.zeros_like(out_ref)

    out_ref[:] += part


def kernel(node_embedding, batch, natoms, W1, b1, W2, b2, W3, b3):
    x2d = node_embedding.reshape(_N, _L * _C)
    bpad = jnp.pad(batch, (0, _NPAD - _N)).reshape(_NBLK, 1, _BLK)
    out = pl.pallas_call(
        _mlp_energy_body,
        grid=(_NBLK,),
        in_specs=[
            pl.BlockSpec(memory_space=pltpu.SMEM),
            pl.BlockSpec((_BLK, _C), lambda i: (i, 0)),
            pl.BlockSpec((1, 1, _BLK), lambda i: (i, 0, 0)),
            pl.BlockSpec((_C, _H), lambda i: (0, 0)),
            pl.BlockSpec((1, _H), lambda i: (0, 0)),
            pl.BlockSpec((_H, _H), lambda i: (0, 0)),
            pl.BlockSpec((1, _H), lambda i: (0, 0)),
            pl.BlockSpec((1, _H), lambda i: (0, 0)),
        ],
        out_specs=pl.BlockSpec((1, _G), lambda i: (0, 0)),
        out_shape=jax.ShapeDtypeStruct((1, _G), jnp.float32),
    )(b3, x2d, bpad, W1, b1.reshape(1, _H), W2,
      b2.reshape(1, _H), W3.reshape(1, _H))
    return out[0]


# speedup vs baseline: 1.5646x; 1.5646x over previous
import jax
import jax.numpy as jnp
from jax.experimental import pallas as pl
from jax.experimental.pallas import tpu as pltpu

_N, _L, _C, _H, _G = 50000, 9, 256, 512, 256
_BLK = 400
_NBLK = _N // _BLK


def _mlp_energy_body(b3_ref, x_hbm, bidx_ref, w1_ref, b1_ref, w2_ref, b2_ref,
                     w3_ref, out_ref, xbuf, sem):
    i = pl.program_id(0)

    def x_copy(blk, slot):
        return pltpu.make_async_copy(
            x_hbm.at[pl.ds(blk * _BLK, _BLK), 0, :], xbuf.at[slot],
            sem.at[slot])

    @pl.when(i == 0)
    def _():
        x_copy(0, 0).start()

    @pl.when(i + 1 < _NBLK)
    def _():
        x_copy(i + 1, (i + 1) % 2).start()

    x_copy(i, i % 2).wait()

    x = xbuf[i % 2].astype(jnp.bfloat16)
    h = jnp.dot(x, w1_ref[:].astype(jnp.bfloat16),
                preferred_element_type=jnp.float32) + b1_ref[:]
    h = h * jax.nn.sigmoid(h)
    h = jnp.dot(h.astype(jnp.bfloat16), w2_ref[:].astype(jnp.bfloat16),
                preferred_element_type=jnp.float32) + b2_ref[:]
    h = h * jax.nn.sigmoid(h)
    e = jnp.sum(h * w3_ref[:], axis=1) + b3_ref[0]
    idx = bidx_ref[0, 0, :]
    onehot = (idx[:, None] == jax.lax.broadcasted_iota(
        jnp.int32, (_BLK, _G), 1)).astype(jnp.float32)
    part = jnp.dot(e[None, :], onehot, preferred_element_type=jnp.float32)

    @pl.when(i == 0)
    def _():
        out_ref[:] = jnp.zeros_like(out_ref)

    out_ref[:] += part


def kernel(node_embedding, batch, natoms, W1, b1, W2, b2, W3, b3):
    bidx = batch.reshape(_NBLK, 1, _BLK)
    out = pl.pallas_call(
        _mlp_energy_body,
        grid=(_NBLK,),
        in_specs=[
            pl.BlockSpec(memory_space=pltpu.SMEM),
            pl.BlockSpec(memory_space=pl.ANY),
            pl.BlockSpec((1, 1, _BLK), lambda i: (i, 0, 0)),
            pl.BlockSpec((_C, _H), lambda i: (0, 0)),
            pl.BlockSpec((1, _H), lambda i: (0, 0)),
            pl.BlockSpec((_H, _H), lambda i: (0, 0)),
            pl.BlockSpec((1, _H), lambda i: (0, 0)),
            pl.BlockSpec((1, _H), lambda i: (0, 0)),
        ],
        out_specs=pl.BlockSpec((1, _G), lambda i: (0, 0)),
        out_shape=jax.ShapeDtypeStruct((1, _G), jnp.float32),
        scratch_shapes=[
            pltpu.VMEM((2, _BLK, _C), jnp.float32),
            pltpu.SemaphoreType.DMA((2,)),
        ],
    )(b3, node_embedding, bidx, W1, b1.reshape(1, _H), W2,
      b2.reshape(1, _H), W3.reshape(1, _H))
    return out[0]


# speedup vs baseline: 1.5651x; 1.0003x over previous
import jax
import jax.numpy as jnp
from jax.experimental import pallas as pl
from jax.experimental.pallas import tpu as pltpu

_N, _L, _C, _H, _G = 50000, 9, 256, 512, 256
_BLK = 400
_NBLK = _N // _BLK


_NSLOT = 3
_KSPL = 5
_SUB = _BLK // _KSPL


def _mlp_energy_body(b3_ref, x_hbm, bidx_ref, w1_ref, b1_ref, w2_ref, b2_ref,
                     w3_ref, out_ref, xbuf, sem):
    i = pl.program_id(0)

    def x_copy(blk, slot, k):
        return pltpu.make_async_copy(
            x_hbm.at[pl.ds(blk * _BLK + k * _SUB, _SUB), 0, :],
            xbuf.at[slot, pl.ds(k * _SUB, _SUB)],
            sem.at[slot, k])

    def start_blk(blk):
        for k in range(_KSPL):
            x_copy(blk, blk % _NSLOT, k).start()

    @pl.when(i == 0)
    def _():
        for b in range(_NSLOT - 1):
            start_blk(b)

    @pl.when(i + _NSLOT - 1 < _NBLK)
    def _():
        start_blk(i + _NSLOT - 1)

    for k in range(_KSPL):
        x_copy(i, i % _NSLOT, k).wait()

    x = xbuf[i % _NSLOT].astype(jnp.bfloat16)
    h = jnp.dot(x, w1_ref[:].astype(jnp.bfloat16),
                preferred_element_type=jnp.float32) + b1_ref[:]
    h = h * jax.nn.sigmoid(h)
    h = jnp.dot(h.astype(jnp.bfloat16), w2_ref[:].astype(jnp.bfloat16),
                preferred_element_type=jnp.float32) + b2_ref[:]
    h = h * jax.nn.sigmoid(h)
    e = jnp.sum(h * w3_ref[:], axis=1) + b3_ref[0]
    idx = bidx_ref[0, 0, :]
    onehot = (idx[:, None] == jax.lax.broadcasted_iota(
        jnp.int32, (_BLK, _G), 1)).astype(jnp.float32)
    part = jnp.dot(e[None, :], onehot, preferred_element_type=jnp.float32)

    @pl.when(i == 0)
    def _():
        out_ref[:] = jnp.zeros_like(out_ref)

    out_ref[:] += part


def kernel(node_embedding, batch, natoms, W1, b1, W2, b2, W3, b3):
    bidx = batch.reshape(_NBLK, 1, _BLK)
    out = pl.pallas_call(
        _mlp_energy_body,
        grid=(_NBLK,),
        in_specs=[
            pl.BlockSpec(memory_space=pltpu.SMEM),
            pl.BlockSpec(memory_space=pl.ANY),
            pl.BlockSpec((1, 1, _BLK), lambda i: (i, 0, 0)),
            pl.BlockSpec((_C, _H), lambda i: (0, 0)),
            pl.BlockSpec((1, _H), lambda i: (0, 0)),
            pl.BlockSpec((_H, _H), lambda i: (0, 0)),
            pl.BlockSpec((1, _H), lambda i: (0, 0)),
            pl.BlockSpec((1, _H), lambda i: (0, 0)),
        ],
        out_specs=pl.BlockSpec((1, _G), lambda i: (0, 0)),
        out_shape=jax.ShapeDtypeStruct((1, _G), jnp.float32),
        scratch_shapes=[
            pltpu.VMEM((_NSLOT, _BLK, _C), jnp.float32),
            pltpu.SemaphoreType.DMA((_NSLOT, _KSPL)),
        ],
    )(b3, node_embedding, bidx, W1, b1.reshape(1, _H), W2,
      b2.reshape(1, _H), W3.reshape(1, _H))
    return out[0]


# speedup vs baseline: 5.2368x; 3.3459x over previous
import jax
import jax.numpy as jnp
from jax.experimental import pallas as pl
from jax.experimental.pallas import tpu as pltpu

_N, _L, _C, _H, _G = 50000, 9, 256, 512, 256
_BLK = 400
_NBLK = _N // _BLK


def _mlp_energy_body(b3_ref, x_ref, bidx_ref, w1_ref, b1_ref, w2_ref, b2_ref,
                     w3_ref, out_ref):
    i = pl.program_id(0)
    x = x_ref[:, :]
    h = jnp.dot(x, w1_ref[:], preferred_element_type=jnp.float32) + b1_ref[:]
    h = h * jax.nn.sigmoid(h)
    h = jnp.dot(h.astype(jnp.bfloat16), w2_ref[:],
                preferred_element_type=jnp.float32) + b2_ref[:]
    h = h * jax.nn.sigmoid(h)
    e = jnp.sum(h * w3_ref[:], axis=1) + b3_ref[0]
    idx = bidx_ref[0, 0, :]
    onehot = (idx[:, None] == jax.lax.broadcasted_iota(
        jnp.int32, (_BLK, _G), 1)).astype(jnp.float32)
    part = jnp.dot(e[None, :], onehot, preferred_element_type=jnp.float32)

    @pl.when(i == 0)
    def _():
        out_ref[:] = jnp.zeros_like(out_ref)

    out_ref[:] += part


def kernel(node_embedding, batch, natoms, W1, b1, W2, b2, W3, b3):
    x_bf = node_embedding[:, 0, :].astype(jnp.bfloat16)
    bidx = batch.reshape(_NBLK, 1, _BLK)
    out = pl.pallas_call(
        _mlp_energy_body,
        grid=(_NBLK,),
        in_specs=[
            pl.BlockSpec(memory_space=pltpu.SMEM),
            pl.BlockSpec((_BLK, _C), lambda i: (i, 0)),
            pl.BlockSpec((1, 1, _BLK), lambda i: (i, 0, 0)),
            pl.BlockSpec((_C, _H), lambda i: (0, 0)),
            pl.BlockSpec((1, _H), lambda i: (0, 0)),
            pl.BlockSpec((_H, _H), lambda i: (0, 0)),
            pl.BlockSpec((1, _H), lambda i: (0, 0)),
            pl.BlockSpec((1, _H), lambda i: (0, 0)),
        ],
        out_specs=pl.BlockSpec((1, _G), lambda i: (0, 0)),
        out_shape=jax.ShapeDtypeStruct((1, _G), jnp.float32),
    )(b3, x_bf, bidx, W1.astype(jnp.bfloat16), b1.reshape(1, _H),
      W2.astype(jnp.bfloat16), b2.reshape(1, _H), W3.reshape(1, _H))
    return out[0]


# speedup vs baseline: 6.7800x; 1.2947x over previous
import jax
import jax.numpy as jnp
from jax.experimental import pallas as pl
from jax.experimental.pallas import tpu as pltpu

_N, _L, _C, _H, _G = 50000, 9, 256, 512, 256
_BLK = 1000
_NBLK = _N // _BLK


def _mlp_energy_body(b3_ref, x_ref, bidx_ref, w1_ref, b1_ref, w2_ref, b2_ref,
                     w3_ref, out_ref):
    i = pl.program_id(0)
    x = x_ref[:, :]
    h = jnp.dot(x, w1_ref[:], preferred_element_type=jnp.float32) + b1_ref[:]
    h = h * jax.nn.sigmoid(h)
    h = jnp.dot(h.astype(jnp.bfloat16), w2_ref[:],
                preferred_element_type=jnp.float32) + b2_ref[:]
    h = h * jax.nn.sigmoid(h)
    e = jnp.sum(h * w3_ref[:], axis=1) + b3_ref[0]
    idx = bidx_ref[0, 0, :]
    onehot = (idx[:, None] == jax.lax.broadcasted_iota(
        jnp.int32, (_BLK, _G), 1)).astype(jnp.float32)
    part = jnp.dot(e[None, :], onehot, preferred_element_type=jnp.float32)

    @pl.when(i == 0)
    def _():
        out_ref[:] = jnp.zeros_like(out_ref)

    out_ref[:] += part


def kernel(node_embedding, batch, natoms, W1, b1, W2, b2, W3, b3):
    x_bf = node_embedding[:, 0, :].astype(jnp.bfloat16)
    bidx = batch.reshape(_NBLK, 1, _BLK)
    out = pl.pallas_call(
        _mlp_energy_body,
        grid=(_NBLK,),
        in_specs=[
            pl.BlockSpec(memory_space=pltpu.SMEM),
            pl.BlockSpec((_BLK, _C), lambda i: (i, 0)),
            pl.BlockSpec((1, 1, _BLK), lambda i: (i, 0, 0)),
            pl.BlockSpec((_C, _H), lambda i: (0, 0)),
            pl.BlockSpec((1, _H), lambda i: (0, 0)),
            pl.BlockSpec((_H, _H), lambda i: (0, 0)),
            pl.BlockSpec((1, _H), lambda i: (0, 0)),
            pl.BlockSpec((1, _H), lambda i: (0, 0)),
        ],
        out_specs=pl.BlockSpec((1, _G), lambda i: (0, 0)),
        out_shape=jax.ShapeDtypeStruct((1, _G), jnp.float32),
    )(b3, x_bf, bidx, W1.astype(jnp.bfloat16), b1.reshape(1, _H),
      W2.astype(jnp.bfloat16), b2.reshape(1, _H), W3.reshape(1, _H))
    return out[0]


# speedup vs baseline: 7.5412x; 1.1123x over previous
import jax
import jax.numpy as jnp
from jax.experimental import pallas as pl
from jax.experimental.pallas import tpu as pltpu

_N, _L, _C, _H, _G = 50000, 9, 256, 512, 256
_BLK = 2000
_NBLK = _N // _BLK


def _mlp_energy_body(b3_ref, x_ref, bidx_ref, w1_ref, b1_ref, w2_ref, b2_ref,
                     w3_ref, out_ref):
    i = pl.program_id(0)
    x = x_ref[:, :]
    h = jnp.dot(x, w1_ref[:], preferred_element_type=jnp.float32) + b1_ref[:]
    h = h * jax.nn.sigmoid(h)
    h = jnp.dot(h.astype(jnp.bfloat16), w2_ref[:],
                preferred_element_type=jnp.float32) + b2_ref[:]
    h = h * jax.nn.sigmoid(h)
    e = jnp.sum(h * w3_ref[:], axis=1) + b3_ref[0]
    idx = bidx_ref[0, 0, :]
    onehot = (idx[:, None] == jax.lax.broadcasted_iota(
        jnp.int32, (_BLK, _G), 1)).astype(jnp.float32)
    part = jnp.dot(e[None, :], onehot, preferred_element_type=jnp.float32)

    @pl.when(i == 0)
    def _():
        out_ref[:] = jnp.zeros_like(out_ref)

    out_ref[:] += part


def kernel(node_embedding, batch, natoms, W1, b1, W2, b2, W3, b3):
    x_bf = node_embedding[:, 0, :].astype(jnp.bfloat16)
    bidx = batch.reshape(_NBLK, 1, _BLK)
    out = pl.pallas_call(
        _mlp_energy_body,
        grid=(_NBLK,),
        in_specs=[
            pl.BlockSpec(memory_space=pltpu.SMEM),
            pl.BlockSpec((_BLK, _C), lambda i: (i, 0)),
            pl.BlockSpec((1, 1, _BLK), lambda i: (i, 0, 0)),
            pl.BlockSpec((_C, _H), lambda i: (0, 0)),
            pl.BlockSpec((1, _H), lambda i: (0, 0)),
            pl.BlockSpec((_H, _H), lambda i: (0, 0)),
            pl.BlockSpec((1, _H), lambda i: (0, 0)),
            pl.BlockSpec((1, _H), lambda i: (0, 0)),
        ],
        out_specs=pl.BlockSpec((1, _G), lambda i: (0, 0)),
        out_shape=jax.ShapeDtypeStruct((1, _G), jnp.float32),
    )(b3, x_bf, bidx, W1.astype(jnp.bfloat16), b1.reshape(1, _H),
      W2.astype(jnp.bfloat16), b2.reshape(1, _H), W3.reshape(1, _H))
    return out[0]
